# baseline (speedup 1.0000x reference)
import functools

import jax
import jax.numpy as jnp
from jax import lax
from jax.experimental import pallas as pl
from jax.experimental.pallas import tpu as pltpu
from jax.experimental.pallas import tpu_sc as plsc

_SPLITS = 4
_LEN_SPLIT = 8
_SPLIT_EMBED = 16
_NUM_EMBEDDING = 1 << _LEN_SPLIT
_NUM_TABLES = 2 * _SPLITS
_N = 425984
_D = _NUM_TABLES * _SPLIT_EMBED

_NC, _NS, _L = 2, 16, 16
_NW = _NC * _NS
_PER_W = _N // _NW
_C = 256
_CHUNKS = _PER_W // _C
_CH = _C * _D
_TAB = _NUM_TABLES * _NUM_EMBEDDING * _SPLIT_EMBED


def _body(x_hbm, tab_hbm, out_hbm, tab_v, x_v, stg_v, sem):
  wid = lax.axis_index("s") * _NC + lax.axis_index("c")
  pltpu.sync_copy(tab_hbm, tab_v)
  pltpu.sync_copy(x_hbm.at[pl.ds(wid * _PER_W, _PER_W)], x_v)

  zeros = jnp.zeros((_L,), jnp.int32)
  four = jnp.full((_L,), 4, jnp.int32)
  maskff0 = jnp.full((_L,), 0xFF0, jnp.int32)
  shr = [jnp.full((_L,), 8 * i - 4, jnp.int32) for i in range(1, _SPLITS)]
  iota16 = lax.iota(jnp.int32, _L)
  ctv = [iota16 + t * _NUM_EMBEDDING * _SPLIT_EMBED
         for t in range(_NUM_TABLES)]
  all_true = jnp.full((_L,), True, jnp.bool_)
  out_base0 = wid * (_PER_W * _D)

  def chunk(g, carry):
    cur = lax.rem(g, 2)
    stg_off = cur * _CH

    @pl.when(g >= 2)
    def _drain():
      pltpu.make_async_copy(
          stg_v.at[pl.ds(stg_off, _CH)],
          out_hbm.at[pl.ds(out_base0, _CH)],
          sem,
      ).wait()

    def group(b, carry2):
      x = x_v[pl.ds(g * _C + b * _L, _L)]
      off0 = stg_off + b * (_L * _D)
      pend = []
      for e in range(_L):
        xb = x.at[jnp.full((_L,), e, jnp.int32)].get(mode="promise_in_bounds")
        neg = xb < zeros
        xa = jnp.abs(xb)
        off_e = off0 + e * _D
        p4s = [lax.shift_left(xa, four) & maskff0]
        for i in range(1, _SPLITS):
          p4s.append(lax.shift_right_arithmetic(xa, shr[i - 1]) & maskff0)
        for t in range(_NUM_TABLES):
          adr = p4s[t % _SPLITS] + ctv[t]
          base = jnp.where(neg, ctv[t], adr) if t < _SPLITS else jnp.where(
              neg, adr, ctv[t])
          val = plsc.load_gather(tab_v, [base], mask=all_true)
          pend.append((off_e + t * _SPLIT_EMBED, val))
          if len(pend) > 8:
            o, v = pend.pop(0)
            stg_v[pl.ds(o, _SPLIT_EMBED)] = v
      for o, v in pend:
        stg_v[pl.ds(o, _SPLIT_EMBED)] = v
      return carry2

    lax.fori_loop(0, _C // _L, group, 0)
    pltpu.async_copy(
        stg_v.at[pl.ds(stg_off, _CH)],
        out_hbm.at[pl.ds(out_base0 + g * _CH, _CH)],
        sem,
    )
    return carry

  lax.fori_loop(0, _CHUNKS, chunk, 0)
  for _ in range(2):
    pltpu.make_async_copy(
        stg_v.at[pl.ds(0, _CH)],
        out_hbm.at[pl.ds(out_base0, _CH)],
        sem,
    ).wait()


_gather = functools.partial(
    pl.kernel,
    out_type=jax.ShapeDtypeStruct((_N * _D,), jnp.float32),
    mesh=plsc.VectorSubcoreMesh(core_axis_name="c", subcore_axis_name="s"),
    compiler_params=pltpu.CompilerParams(
        needs_layout_passes=False, use_tc_tiling_on_sc=False),
    scratch_types=[
        pltpu.VMEM((_TAB,), jnp.float32),
        pltpu.VMEM((_PER_W,), jnp.int32),
        pltpu.VMEM((2 * _CH,), jnp.float32),
        pltpu.SemaphoreType.DMA,
    ],
)(_body)


@jax.jit
def kernel(X, tables):
  out = _gather(X, tables.reshape(-1))
  return out.reshape(_N, _D)


# speedup vs baseline: 1.0398x; 1.0398x over previous
import functools

import jax
import jax.numpy as jnp
from jax import lax
from jax.experimental import pallas as pl
from jax.experimental.pallas import tpu as pltpu
from jax.experimental.pallas import tpu_sc as plsc

_SPLITS = 4
_LEN_SPLIT = 8
_SPLIT_EMBED = 16
_NUM_EMBEDDING = 1 << _LEN_SPLIT
_NUM_TABLES = 2 * _SPLITS
_N = 425984
_D = _NUM_TABLES * _SPLIT_EMBED

_NC, _NS, _L = 2, 16, 16
_NW = _NC * _NS
_PER_W = _N // _NW
_C = 256
_CHUNKS = _PER_W // _C
_CH = _C * _D
_TAB = _NUM_TABLES * _NUM_EMBEDDING * _SPLIT_EMBED


def _body(x_hbm, tab_hbm, out_hbm, tab_v, x_v, stg_v, sem):
  wid = lax.axis_index("s") * _NC + lax.axis_index("c")
  pltpu.sync_copy(tab_hbm, tab_v)
  pltpu.sync_copy(x_hbm.at[pl.ds(wid * _PER_W, _PER_W)], x_v)

  zeros = jnp.zeros((_L,), jnp.int32)
  four = jnp.full((_L,), 4, jnp.int32)
  maskff0 = jnp.full((_L,), 0xFF0, jnp.int32)
  shr = [jnp.full((_L,), 8 * i - 4, jnp.int32) for i in range(1, _SPLITS)]
  iota16 = lax.iota(jnp.int32, _L)
  ctv = [iota16 + t * _NUM_EMBEDDING * _SPLIT_EMBED
         for t in range(_NUM_TABLES)]
  offs = [jnp.full((_L,), j, jnp.int32) for j in range(_L)]
  all_true = jnp.full((_L,), True, jnp.bool_)
  out_base0 = wid * (_PER_W * _D)

  def chunk(g, carry):
    cur = lax.rem(g, 2)
    stg_off = cur * _CH

    @pl.when(g >= 2)
    def _drain():
      pltpu.make_async_copy(
          stg_v.at[pl.ds(stg_off, _CH)],
          out_hbm.at[pl.ds(out_base0, _CH)],
          sem,
      ).wait()

    def group(b, carry2):
      xs = [x_v[pl.ds(g * _C + b * (2 * _L) + k * _L, _L)] for k in range(2)]
      off0 = stg_off + b * (2 * _L * _D)
      pend = []
      for e in range(2 * _L):
        xb = xs[e // _L].at[offs[e % _L]].get(mode="promise_in_bounds")
        neg = xb < zeros
        xa = jnp.abs(xb)
        off_e = off0 + e * _D
        p4s = [lax.shift_left(xa, four) & maskff0]
        for i in range(1, _SPLITS):
          p4s.append(lax.shift_right_arithmetic(xa, shr[i - 1]) & maskff0)
        for t in range(_NUM_TABLES):
          adr = p4s[t % _SPLITS] + ctv[t]
          base = jnp.where(neg, ctv[t], adr) if t < _SPLITS else jnp.where(
              neg, adr, ctv[t])
          val = plsc.load_gather(tab_v, [base], mask=all_true)
          pend.append((off_e + t * _SPLIT_EMBED, val))
          if len(pend) > 6:
            o, v = pend.pop(0)
            stg_v[pl.ds(o, _SPLIT_EMBED)] = v
      for o, v in pend:
        stg_v[pl.ds(o, _SPLIT_EMBED)] = v
      return carry2

    lax.fori_loop(0, _C // (2 * _L), group, 0)
    pltpu.async_copy(
        stg_v.at[pl.ds(stg_off, _CH)],
        out_hbm.at[pl.ds(out_base0 + g * _CH, _CH)],
        sem,
    )
    return carry

  lax.fori_loop(0, _CHUNKS, chunk, 0)
  for _ in range(2):
    pltpu.make_async_copy(
        stg_v.at[pl.ds(0, _CH)],
        out_hbm.at[pl.ds(out_base0, _CH)],
        sem,
    ).wait()


_gather = functools.partial(
    pl.kernel,
    out_type=jax.ShapeDtypeStruct((_N * _D,), jnp.float32),
    mesh=plsc.VectorSubcoreMesh(core_axis_name="c", subcore_axis_name="s"),
    compiler_params=pltpu.CompilerParams(
        needs_layout_passes=False, use_tc_tiling_on_sc=False),
    scratch_types=[
        pltpu.VMEM((_TAB,), jnp.float32),
        pltpu.VMEM((_PER_W,), jnp.int32),
        pltpu.VMEM((2 * _CH,), jnp.float32),
        pltpu.SemaphoreType.DMA,
    ],
)(_body)


@jax.jit
def kernel(X, tables):
  out = _gather(X, tables.reshape(-1))
  return out.reshape(_N, _D)
